# Initial kernel scaffold; baseline (speedup 1.0000x reference)
import jax
import jax.numpy as jnp
from jax.experimental import pallas as pl


def kernel(x, feature_matrix):
    raise NotImplementedError("write your pallas kernel here")


# speedup vs baseline: 4.6198x; 4.6198x over previous
import functools

import jax
import jax.numpy as jnp
from jax import lax
from jax.experimental import pallas as pl
from jax.experimental.pallas import tpu as pltpu
from jax.experimental.pallas import tpu_sc as plsc

_NC = 2
_NS = 16
_NW = _NC * _NS
_CH = 128
_G = 10


@functools.partial(jax.jit, static_argnames=("n_ch", "d"))
def _emb_lookup(xf, feature_matrix, n_ch, d):
    n = _NW * n_ch * _CH
    n_sup = n_ch // _G
    assert n_ch % _G == 0 and n_sup % 2 == 0
    blk = _G * _CH
    mesh = plsc.VectorSubcoreMesh(core_axis_name="c", subcore_axis_name="s")

    @functools.partial(
        pl.kernel,
        mesh=mesh,
        compiler_params=pltpu.CompilerParams(use_tc_tiling_on_sc=False),
        out_type=jax.ShapeDtypeStruct((n, d), jnp.float32),
        scratch_types=[
            pltpu.VMEM((n_ch, _CH), jnp.int32),
            pltpu.VMEM((2, blk, d), jnp.float32),
            pltpu.SemaphoreType.DMA,
            pltpu.SemaphoreType.DMA,
            pltpu.SemaphoreType.DMA,
        ],
    )
    def emb(x_hbm, tab_hbm, out_hbm, idx_v, rows_v, gsem, osem0, osem1):
        wid = lax.axis_index("s") * _NC + lax.axis_index("c")
        base = wid * (n_ch * _CH)
        pltpu.sync_copy(x_hbm.at[wid], idx_v)

        def half(t, s, slot, osem):
            for g in range(_G):
                pltpu.async_copy(
                    tab_hbm.at[idx_v.at[s * _G + g]],
                    rows_v.at[slot, pl.ds(g * _CH, _CH)],
                    gsem,
                )
            for g in range(_G):
                pltpu.make_async_copy(
                    tab_hbm.at[idx_v.at[g]],
                    rows_v.at[slot, pl.ds(g * _CH, _CH)],
                    gsem,
                ).wait()

            @pl.when(t >= 1)
            def _():
                pltpu.make_async_copy(
                    rows_v.at[slot], out_hbm.at[pl.ds(base, blk)], osem
                ).wait()

            pltpu.async_copy(
                rows_v.at[slot], out_hbm.at[pl.ds(base + s * blk, blk)], osem
            )

        def body(t, carry):
            half(t, 2 * t, 0, osem0)
            half(t, 2 * t + 1, 1, osem1)
            return carry

        lax.fori_loop(0, n_sup // 2, body, 0)
        pltpu.make_async_copy(
            rows_v.at[0], out_hbm.at[pl.ds(base, blk)], osem0
        ).wait()
        pltpu.make_async_copy(
            rows_v.at[1], out_hbm.at[pl.ds(base, blk)], osem1
        ).wait()

    return emb(xf, feature_matrix)


def kernel(x, feature_matrix):
    b, s = x.shape
    v, d = feature_matrix.shape
    n = b * s
    assert n % (_NW * _CH) == 0
    n_ch = n // (_NW * _CH)
    xf = x.reshape(_NW, n_ch, _CH).astype(jnp.int32)
    out = _emb_lookup(xf, feature_matrix, n_ch, d)
    return out.reshape(b, s, d)
